# Initial kernel scaffold; baseline (speedup 1.0000x reference)
import jax
import jax.numpy as jnp
from jax.experimental import pallas as pl


def kernel(object_features, masks, pos_embed, W1, b1, W2, b2):
    raise NotImplementedError("write your pallas kernel here")


# speedup vs baseline: 1.8540x; 1.8540x over previous
import jax
import jax.numpy as jnp
from jax.experimental import pallas as pl

_HIGH = jax.lax.Precision.HIGHEST


def _prep_kernel(x_ref, w1_ref, b1_ref, feat_ref, pos_ref):
    n_feat = feat_ref.shape[0]
    y = jax.lax.dot(x_ref[...], w1_ref[...], precision=_HIGH)
    feat_ref[...] = y[:n_feat]
    pos_ref[...] = y[n_feat:] + b1_ref[...]


def _decode_kernel(feat_ref, m_ref, pos_ref, w2_ref, wa_ref, b2_ref, out_ref):
    k_slots = feat_ref.shape[1]
    topk = 8
    feat = feat_ref[0]
    m = m_ref[0]
    pos = pos_ref[...]

    mi = m[:, None, :]
    mj = m[None, :, :]
    ii = jax.lax.broadcasted_iota(jnp.int32, (k_slots, k_slots, 1), 0)
    jj = jax.lax.broadcasted_iota(jnp.int32, (k_slots, k_slots, 1), 1)
    beats = (mj > mi) | ((mj == mi) & (jj < ii))
    rank = jnp.sum(beats.astype(jnp.int32), axis=1)
    sel = rank < topk

    h_all = jax.nn.gelu(feat[:, None, :] + pos[None, :, :])

    alpha = jnp.sum(h_all * wa_ref[...][None, :, :], axis=-1)
    a = jnp.where(sel, alpha, -jnp.inf)
    amax = jnp.max(a, axis=0, keepdims=True)
    e = jnp.where(sel, jnp.exp(a - amax), 0.0)
    w = e / jnp.sum(e, axis=0, keepdims=True)

    hbar = jnp.sum(h_all * w[:, :, None], axis=0)
    out_ref[0] = jax.lax.dot(hbar, w2_ref[...], precision=_HIGH) + b2_ref[...]


def kernel(object_features, masks, pos_embed, W1, b1, W2, b2):
    b, k_slots, d = object_features.shape
    p = pos_embed.shape[1]
    h = W1.shape[1]
    out_dim = W2.shape[1] - 1

    x = jnp.concatenate([object_features.reshape(b * k_slots, d), pos_embed[0]], axis=0)
    feat_proj, pos_base = pl.pallas_call(
        _prep_kernel,
        out_shape=(
            jax.ShapeDtypeStruct((b * k_slots, h), jnp.float32),
            jax.ShapeDtypeStruct((p, h), jnp.float32),
        ),
    )(x, W1, b1.reshape(1, h))

    feat_proj = feat_proj.reshape(b, k_slots, h)
    w2_dec = W2[:, :out_dim]
    w_alpha = W2[:, out_dim].reshape(1, h)
    b2_dec = b2[:out_dim].reshape(1, out_dim)

    out = pl.pallas_call(
        _decode_kernel,
        grid=(b,),
        in_specs=[
            pl.BlockSpec((1, k_slots, h), lambda i: (i, 0, 0)),
            pl.BlockSpec((1, k_slots, p), lambda i: (i, 0, 0)),
            pl.BlockSpec((p, h), lambda i: (0, 0)),
            pl.BlockSpec((h, out_dim), lambda i: (0, 0)),
            pl.BlockSpec((1, h), lambda i: (0, 0)),
            pl.BlockSpec((1, out_dim), lambda i: (0, 0)),
        ],
        out_specs=pl.BlockSpec((1, p, out_dim), lambda i: (i, 0, 0)),
        out_shape=jax.ShapeDtypeStruct((b, p, out_dim), jnp.float32),
    )(feat_proj, masks, pos_base, w2_dec, w_alpha, b2_dec)
    return out


# speedup vs baseline: 2.7606x; 1.4890x over previous
import jax
import jax.numpy as jnp
from jax.experimental import pallas as pl

_HIGH = jax.lax.Precision.HIGHEST


def _prep_kernel(x_ref, w1_ref, b1_ref, feat_ref, pos_ref):
    n_feat = feat_ref.shape[0]
    y = jax.lax.dot(x_ref[...], w1_ref[...], precision=_HIGH)
    feat_ref[...] = y[:n_feat].astype(jnp.bfloat16)
    pos_ref[...] = (y[n_feat:] + b1_ref[...]).astype(jnp.bfloat16)


def _decode_kernel(feat_ref, m_ref, pos_ref, w2_ref, wa_ref, b2_ref, out_ref):
    k_slots = feat_ref.shape[1]
    topk = 8
    feat = feat_ref[0]
    m = m_ref[0]
    pos = pos_ref[...]

    mi = m[:, None, :]
    mj = m[None, :, :]
    ii = jax.lax.broadcasted_iota(jnp.int32, (k_slots, k_slots, 1), 0)
    jj = jax.lax.broadcasted_iota(jnp.int32, (k_slots, k_slots, 1), 1)
    beats = (mj > mi) | ((mj == mi) & (jj < ii))
    rank = jnp.sum(beats.astype(jnp.int32), axis=1)
    sel = rank < topk

    h_all = jax.nn.gelu(feat[:, None, :] + pos[None, :, :])

    alpha = jnp.sum((h_all * wa_ref[...][None, :, :]).astype(jnp.float32),
                    axis=-1)
    a = jnp.where(sel, alpha, -jnp.inf)
    amax = jnp.max(a, axis=0, keepdims=True)
    e = jnp.where(sel, jnp.exp(a - amax), 0.0)
    w = e / jnp.sum(e, axis=0, keepdims=True)

    hbar = jnp.sum(h_all.astype(jnp.float32) * w[:, :, None], axis=0)
    out_ref[0] = jax.lax.dot(
        hbar.astype(jnp.bfloat16), w2_ref[...],
        preferred_element_type=jnp.float32) + b2_ref[...]


def kernel(object_features, masks, pos_embed, W1, b1, W2, b2):
    b, k_slots, d = object_features.shape
    p = pos_embed.shape[1]
    h = W1.shape[1]
    out_dim = W2.shape[1] - 1

    x = jnp.concatenate([object_features.reshape(b * k_slots, d), pos_embed[0]], axis=0)
    feat_proj, pos_base = pl.pallas_call(
        _prep_kernel,
        out_shape=(
            jax.ShapeDtypeStruct((b * k_slots, h), jnp.bfloat16),
            jax.ShapeDtypeStruct((p, h), jnp.bfloat16),
        ),
    )(x, W1, b1.reshape(1, h))

    feat_proj = feat_proj.reshape(b, k_slots, h)
    w2_dec = W2[:, :out_dim].astype(jnp.bfloat16)
    w_alpha = W2[:, out_dim].reshape(1, h).astype(jnp.bfloat16)
    b2_dec = b2[:out_dim].reshape(1, out_dim)

    out = pl.pallas_call(
        _decode_kernel,
        grid=(b,),
        in_specs=[
            pl.BlockSpec((1, k_slots, h), lambda i: (i, 0, 0)),
            pl.BlockSpec((1, k_slots, p), lambda i: (i, 0, 0)),
            pl.BlockSpec((p, h), lambda i: (0, 0)),
            pl.BlockSpec((h, out_dim), lambda i: (0, 0)),
            pl.BlockSpec((1, h), lambda i: (0, 0)),
            pl.BlockSpec((1, out_dim), lambda i: (0, 0)),
        ],
        out_specs=pl.BlockSpec((1, p, out_dim), lambda i: (i, 0, 0)),
        out_shape=jax.ShapeDtypeStruct((b, p, out_dim), jnp.float32),
    )(feat_proj, masks, pos_base, w2_dec, w_alpha, b2_dec)
    return out


# speedup vs baseline: 3.0965x; 1.1217x over previous
import jax
import jax.numpy as jnp
from jax.experimental import pallas as pl

_HIGH = jax.lax.Precision.HIGHEST


def _prep_kernel(x_ref, w1_ref, b1_ref, feat_ref, pos_ref):
    n_feat = feat_ref.shape[0]
    y = jax.lax.dot(x_ref[...], w1_ref[...], precision=_HIGH)
    feat_ref[...] = y[:n_feat].astype(jnp.bfloat16)
    pos_ref[...] = (y[n_feat:] + b1_ref[...]).astype(jnp.bfloat16)


def _decode_kernel(feat_ref, m_ref, pos_ref, w2_ref, wa_ref, b2_ref, out_ref):
    k_slots = feat_ref.shape[1]
    topk = 8
    feat = feat_ref[0]
    m = m_ref[0]
    pos = pos_ref[...]

    mi = m[:, None, :]
    mj = m[None, :, :]
    ii = jax.lax.broadcasted_iota(jnp.int32, (k_slots, k_slots, 1), 0)
    jj = jax.lax.broadcasted_iota(jnp.int32, (k_slots, k_slots, 1), 1)
    beats = (mj > mi) | ((mj == mi) & (jj < ii))
    rank = jnp.sum(beats.astype(jnp.int32), axis=1)
    sel = rank < topk

    x = feat[:, None, :] + pos[None, :, :]
    c0 = jnp.bfloat16(0.7978845608028654)
    c1 = jnp.bfloat16(0.7978845608028654 * 0.044715)
    t = x * x
    y = x * (c0 + c1 * t)
    r = jnp.bfloat16(0.5) * x
    h_all = r + r * jnp.tanh(y)

    alpha = jnp.sum((h_all * wa_ref[...][None, :, :]).astype(jnp.float32),
                    axis=-1)
    a = jnp.where(sel, alpha, -jnp.inf)
    amax = jnp.max(a, axis=0, keepdims=True)
    e = jnp.where(sel, jnp.exp(a - amax), 0.0)
    w = e / jnp.sum(e, axis=0, keepdims=True)

    wb = w.astype(jnp.bfloat16)
    hbar = jnp.sum(h_all * wb[:, :, None], axis=0)
    out_ref[0] = jax.lax.dot(
        hbar, w2_ref[...],
        preferred_element_type=jnp.float32) + b2_ref[...]


def kernel(object_features, masks, pos_embed, W1, b1, W2, b2):
    b, k_slots, d = object_features.shape
    p = pos_embed.shape[1]
    h = W1.shape[1]
    out_dim = W2.shape[1] - 1

    x = jnp.concatenate([object_features.reshape(b * k_slots, d), pos_embed[0]], axis=0)
    feat_proj, pos_base = pl.pallas_call(
        _prep_kernel,
        out_shape=(
            jax.ShapeDtypeStruct((b * k_slots, h), jnp.bfloat16),
            jax.ShapeDtypeStruct((p, h), jnp.bfloat16),
        ),
    )(x, W1, b1.reshape(1, h))

    feat_proj = feat_proj.reshape(b, k_slots, h)
    w2_dec = W2[:, :out_dim].astype(jnp.bfloat16)
    w_alpha = W2[:, out_dim].reshape(1, h).astype(jnp.bfloat16)
    b2_dec = b2[:out_dim].reshape(1, out_dim)

    out = pl.pallas_call(
        _decode_kernel,
        grid=(b,),
        in_specs=[
            pl.BlockSpec((1, k_slots, h), lambda i: (i, 0, 0)),
            pl.BlockSpec((1, k_slots, p), lambda i: (i, 0, 0)),
            pl.BlockSpec((p, h), lambda i: (0, 0)),
            pl.BlockSpec((h, out_dim), lambda i: (0, 0)),
            pl.BlockSpec((1, h), lambda i: (0, 0)),
            pl.BlockSpec((1, out_dim), lambda i: (0, 0)),
        ],
        out_specs=pl.BlockSpec((1, p, out_dim), lambda i: (i, 0, 0)),
        out_shape=jax.ShapeDtypeStruct((b, p, out_dim), jnp.float32),
    )(feat_proj, masks, pos_base, w2_dec, w_alpha, b2_dec)
    return out


# speedup vs baseline: 3.8392x; 1.2399x over previous
import jax
import jax.numpy as jnp
from jax.experimental import pallas as pl


def _prep_kernel(x_ref, w1_ref, b1_ref, w2_ref, feat_ref, pos_ref,
                 w2d_ref, wa_ref):
    n_feat = feat_ref.shape[0]
    out_dim = w2d_ref.shape[1]
    y = jax.lax.dot(x_ref[...], w1_ref[...], precision=jax.lax.Precision.HIGHEST)
    feat_ref[...] = y[:n_feat].astype(jnp.bfloat16)
    pos_ref[...] = (y[n_feat:] + b1_ref[...]).astype(jnp.bfloat16)
    w2d_ref[...] = w2_ref[:, :out_dim].astype(jnp.bfloat16)
    wa_ref[...] = w2_ref[:, w2_ref.shape[1] - 128:].astype(jnp.bfloat16)


def _decode_kernel(feat_ref, m_ref, pos_ref, w2_ref, wa_ref, b2_ref, out_ref):
    k_slots = feat_ref.shape[1]
    topk = 8
    feat = feat_ref[0]
    m = m_ref[0]
    pos = pos_ref[...]

    mi = m[:, None, :]
    mj = m[None, :, :]
    ii = jax.lax.broadcasted_iota(jnp.int32, (k_slots, k_slots, 1), 0)
    jj = jax.lax.broadcasted_iota(jnp.int32, (k_slots, k_slots, 1), 1)
    beats = (mj > mi) | ((mj == mi) & (jj < ii))
    rank = jnp.sum(beats.astype(jnp.int32), axis=1)
    sel = rank < topk

    x = feat[:, None, :] + pos[None, :, :]
    c0 = jnp.bfloat16(0.7978845608028654)
    c1 = jnp.bfloat16(0.7978845608028654 * 0.044715)
    t = x * x
    y = x * (c0 + c1 * t)
    r = jnp.bfloat16(0.5) * x
    h_all = r + r * jnp.tanh(y)

    p_dim, h_dim = pos.shape
    a3 = jax.lax.dot(h_all.reshape(k_slots * p_dim, h_dim), wa_ref[...],
                     preferred_element_type=jnp.float32)
    alpha = a3[:, 127].reshape(k_slots, p_dim)

    a = jnp.where(sel, alpha, -jnp.inf)
    amax = jnp.max(a, axis=0, keepdims=True)
    e = jnp.where(sel, jnp.exp(a - amax), 0.0)
    w = e / jnp.sum(e, axis=0, keepdims=True)

    wb = w.astype(jnp.bfloat16)
    hw = h_all * wb[:, :, None]
    terms = [hw[k] for k in range(k_slots)]
    while len(terms) > 1:
        terms = [terms[i] + terms[i + 1] for i in range(0, len(terms), 2)]
    hbar = terms[0]

    out_ref[0] = jax.lax.dot(
        hbar, w2_ref[...],
        preferred_element_type=jnp.float32) + b2_ref[...]


def kernel(object_features, masks, pos_embed, W1, b1, W2, b2):
    b, k_slots, d = object_features.shape
    p = pos_embed.shape[1]
    h = W1.shape[1]
    out_dim = W2.shape[1] - 1

    x = jnp.concatenate([object_features.reshape(b * k_slots, d), pos_embed[0]], axis=0)
    feat_proj, pos_base, w2_dec, w_alpha = pl.pallas_call(
        _prep_kernel,
        out_shape=(
            jax.ShapeDtypeStruct((b * k_slots, h), jnp.bfloat16),
            jax.ShapeDtypeStruct((p, h), jnp.bfloat16),
            jax.ShapeDtypeStruct((h, out_dim), jnp.bfloat16),
            jax.ShapeDtypeStruct((h, 128), jnp.bfloat16),
        ),
    )(x, W1, b1.reshape(1, h), W2)

    feat_proj = feat_proj.reshape(b, k_slots, h)
    b2_dec = b2[:out_dim].reshape(1, out_dim)

    out = pl.pallas_call(
        _decode_kernel,
        grid=(b,),
        in_specs=[
            pl.BlockSpec((1, k_slots, h), lambda i: (i, 0, 0)),
            pl.BlockSpec((1, k_slots, p), lambda i: (i, 0, 0)),
            pl.BlockSpec((p, h), lambda i: (0, 0)),
            pl.BlockSpec((h, out_dim), lambda i: (0, 0)),
            pl.BlockSpec((h, 128), lambda i: (0, 0)),
            pl.BlockSpec((1, out_dim), lambda i: (0, 0)),
        ],
        out_specs=pl.BlockSpec((1, p, out_dim), lambda i: (i, 0, 0)),
        out_shape=jax.ShapeDtypeStruct((b, p, out_dim), jnp.float32),
    )(feat_proj, masks, pos_base, w2_dec, w_alpha, b2_dec)
    return out
